# baseline (speedup 1.0000x reference)
import jax
import jax.numpy as jnp
from jax import lax
from jax.experimental import pallas as pl
from jax.experimental.pallas import tpu as pltpu

N = 4096
D = 128
K = 50
RB = 512
NB = N // RB
KEY_HI = 0x3F8147AE
BS_ITERS = 31


def _keys(s):
    k = lax.bitcast_convert_type(s, jnp.int32)
    return jnp.where(k >= 0, k, k ^ jnp.int32(0x7FFFFFFF))


def _unkeys(k):
    return lax.bitcast_convert_type(
        jnp.where(k >= 0, k, k ^ jnp.int32(0x7FFFFFFF)), jnp.float32)


def _dot_t(a, b):
    return lax.dot_general(a, b, (((1,), (1,)), ((), ())),
                           preferred_element_type=jnp.float32)


def _fused_kernel(w_ref, q_ref, ws_ref, bs_ref, wm_ref, bm_ref,
                  wt_ref, bt_ref, wc_ref, bc_ref, out_ref,
                  emb_s, tsub_s, tlane_s):
    p = pl.program_id(0)

    @pl.when(p == 0)
    def _embed():
        q = q_ref[...]
        structural = _dot_t(q, ws_ref[...]) + bs_ref[...]
        semantic = _dot_t(q, wm_ref[...]) + bm_ref[...]
        statistical = _dot_t(q, wt_ref[...]) + bt_ref[...]
        content = _dot_t(q, wc_ref[...]) + bc_ref[...]
        w = w_ref[...]
        weighted = (w[0:1, 0:1] * structural + w[0:1, 1:2] * semantic
                    + w[0:1, 2:3] * statistical + w[0:1, 3:4] * content)
        nrm = jnp.sqrt(jnp.sum(weighted * weighted, axis=1, keepdims=True))
        emb_s[...] = weighted / jnp.maximum(nrm, 1e-12)

    @pl.when((p >= 1) & (p <= NB))
    def _thresh():
        rb = p - 1
        embb = emb_s[pl.ds(rb * RB, RB), :]
        k = _keys(_dot_t(embb, emb_s[...]))

        def body(_, carry):
            lo, hi = carry
            mid = lax.shift_right_arithmetic(lo + hi, 1)
            cnt = jnp.count_nonzero(k >= mid, axis=1, keepdims=True)
            pred = cnt >= K
            return jnp.where(pred, mid, lo), jnp.where(pred, hi, mid)

        lo0 = jnp.full((RB, 1), -(KEY_HI + 1), dtype=jnp.int32)
        hi0 = jnp.full((RB, 1), KEY_HI, dtype=jnp.int32)
        lo, _ = lax.fori_loop(0, BS_ITERS, body, (lo0, hi0), unroll=8)
        t = _unkeys(lo)
        tsub_s[pl.ds(rb * RB, RB), :] = t
        tlane_s[:, pl.ds(rb * RB, RB)] = t.T

    @pl.when(p > NB)
    def _fill():
        rb = p - (NB + 1)
        embb = emb_s[pl.ds(rb * RB, RB), :]
        s = _dot_t(embb, emb_s[...])
        tsub = tsub_s[pl.ds(rb * RB, RB), :]
        keep = s >= jnp.minimum(tsub, tlane_s[...])
        out_ref[...] = jnp.where(keep, s, 0.0)
        strip = out_ref[:, pl.ds(rb * RB, RB)]
        eye = (lax.broadcasted_iota(jnp.int32, (RB, RB), 0)
               == lax.broadcasted_iota(jnp.int32, (RB, RB), 1))
        out_ref[:, pl.ds(rb * RB, RB)] = jnp.where(eye, 1.0, strip)


@jax.jit
def kernel(query_embeddings, similarity_weights, W_struct, b_struct,
           W_sem, b_sem, W_stat, b_stat, W_cont, b_cont):
    w = jax.nn.softmax(similarity_weights, axis=0).reshape(1, 4)
    biases = [b.reshape(1, D) for b in (b_struct, b_sem, b_stat, b_cont)]

    def whole(shape):
        return pl.BlockSpec(shape, lambda i: tuple(0 for _ in shape))

    M = pl.pallas_call(
        _fused_kernel,
        grid=(2 * NB + 1,),
        in_specs=[whole((1, 4)), whole((N, D)),
                  whole((D, D)), whole((1, D)),
                  whole((D, D)), whole((1, D)),
                  whole((D, D)), whole((1, D)),
                  whole((D, D)), whole((1, D))],
        out_specs=pl.BlockSpec(
            (RB, N), lambda i: (jnp.maximum(i - (NB + 1), 0), 0)),
        out_shape=jax.ShapeDtypeStruct((N, N), jnp.float32),
        scratch_shapes=[pltpu.VMEM((N, D), jnp.float32),
                        pltpu.VMEM((N, 1), jnp.float32),
                        pltpu.VMEM((1, N), jnp.float32)],
        compiler_params=pltpu.CompilerParams(
            dimension_semantics=("arbitrary",)),
    )(w, query_embeddings, W_struct, biases[0], W_sem, biases[1],
      W_stat, biases[2], W_cont, biases[3])
    return M


# speedup vs baseline: 1.0140x; 1.0140x over previous
import jax
import jax.numpy as jnp
from jax import lax
from jax.experimental import pallas as pl
from jax.experimental.pallas import tpu as pltpu

N = 4096
D = 128
K = 50
RB = 512
NB = N // RB
KEY_HI = 0x3F8147AE
BS_ITERS = 31


def _keys(s):
    k = lax.bitcast_convert_type(s, jnp.int32)
    return jnp.where(k >= 0, k, k ^ jnp.int32(0x7FFFFFFF))


def _unkeys(k):
    return lax.bitcast_convert_type(
        jnp.where(k >= 0, k, k ^ jnp.int32(0x7FFFFFFF)), jnp.float32)


def _dot_t(a, b):
    return lax.dot_general(a, b, (((1,), (1,)), ((), ())),
                           preferred_element_type=jnp.float32)


def _fused_kernel(w_ref, q_ref, ws_ref, bs_ref, wm_ref, bm_ref,
                  wt_ref, bt_ref, wc_ref, bc_ref, out_ref,
                  emb_s, tsub_s, tlane_s):
    p = pl.program_id(0)

    @pl.when(p == 0)
    def _embed():
        q = q_ref[...]
        structural = _dot_t(q, ws_ref[...]) + bs_ref[...]
        semantic = _dot_t(q, wm_ref[...]) + bm_ref[...]
        statistical = _dot_t(q, wt_ref[...]) + bt_ref[...]
        content = _dot_t(q, wc_ref[...]) + bc_ref[...]
        w = w_ref[...]
        weighted = (w[0:1, 0:1] * structural + w[0:1, 1:2] * semantic
                    + w[0:1, 2:3] * statistical + w[0:1, 3:4] * content)
        nrm = jnp.sqrt(jnp.sum(weighted * weighted, axis=1, keepdims=True))
        emb_s[...] = weighted / jnp.maximum(nrm, 1e-12)

    @pl.when((p >= 1) & (p <= NB))
    def _thresh():
        rb = p - 1
        embb = emb_s[pl.ds(rb * RB, RB), :]
        k = _keys(_dot_t(embb, emb_s[...]))

        def body(_, carry):
            lo, hi = carry
            mid = lax.shift_right_arithmetic(lo + hi, 1)
            cnt = jnp.count_nonzero(k >= mid, axis=1, keepdims=True)
            pred = cnt >= K
            return jnp.where(pred, mid, lo), jnp.where(pred, hi, mid)

        lo0 = jnp.full((RB, 1), -(KEY_HI + 1), dtype=jnp.int32)
        hi0 = jnp.full((RB, 1), KEY_HI, dtype=jnp.int32)
        lo, _ = lax.fori_loop(0, BS_ITERS, body, (lo0, hi0), unroll=31)
        t = _unkeys(lo)
        tsub_s[pl.ds(rb * RB, RB), :] = t
        tlane_s[:, pl.ds(rb * RB, RB)] = t.T

    @pl.when(p > NB)
    def _fill():
        rb = p - (NB + 1)
        embb = emb_s[pl.ds(rb * RB, RB), :]
        s = _dot_t(embb, emb_s[...])
        tsub = tsub_s[pl.ds(rb * RB, RB), :]
        keep = s >= jnp.minimum(tsub, tlane_s[...])
        out_ref[...] = jnp.where(keep, s, 0.0)
        strip = out_ref[:, pl.ds(rb * RB, RB)]
        eye = (lax.broadcasted_iota(jnp.int32, (RB, RB), 0)
               == lax.broadcasted_iota(jnp.int32, (RB, RB), 1))
        out_ref[:, pl.ds(rb * RB, RB)] = jnp.where(eye, 1.0, strip)


@jax.jit
def kernel(query_embeddings, similarity_weights, W_struct, b_struct,
           W_sem, b_sem, W_stat, b_stat, W_cont, b_cont):
    w = jax.nn.softmax(similarity_weights, axis=0).reshape(1, 4)
    biases = [b.reshape(1, D) for b in (b_struct, b_sem, b_stat, b_cont)]

    def whole(shape):
        return pl.BlockSpec(shape, lambda i: tuple(0 for _ in shape))

    M = pl.pallas_call(
        _fused_kernel,
        grid=(2 * NB + 1,),
        in_specs=[whole((1, 4)), whole((N, D)),
                  whole((D, D)), whole((1, D)),
                  whole((D, D)), whole((1, D)),
                  whole((D, D)), whole((1, D)),
                  whole((D, D)), whole((1, D))],
        out_specs=pl.BlockSpec(
            (RB, N), lambda i: (jnp.maximum(i - (NB + 1), 0), 0)),
        out_shape=jax.ShapeDtypeStruct((N, N), jnp.float32),
        scratch_shapes=[pltpu.VMEM((N, D), jnp.float32),
                        pltpu.VMEM((N, 1), jnp.float32),
                        pltpu.VMEM((1, N), jnp.float32)],
        compiler_params=pltpu.CompilerParams(
            dimension_semantics=("arbitrary",)),
    )(w, query_embeddings, W_struct, biases[0], W_sem, biases[1],
      W_stat, biases[2], W_cont, biases[3])
    return M
